# baseline (speedup 1.0000x reference)
import functools

import jax
import jax.numpy as jnp
from jax import lax
from jax.experimental import pallas as pl
from jax.experimental.pallas import tpu as pltpu
from jax.experimental.pallas import tpu_sc as plsc

_SPAD = 56
_IPAD = 64


def _sc_geometry():
    try:
        info = plsc.get_sparse_core_info()
        return info.num_cores, info.num_subcores
    except Exception:
        return 2, 16


@functools.lru_cache(maxsize=None)
def _make_gather(B0, S, D, NC, NS):
    NW = NC * NS
    b_per_w = B0 // NW
    NBUF = 4
    n_steps = b_per_w // NBUF
    assert b_per_w == NBUF * n_steps and n_steps >= 2
    mesh = plsc.VectorSubcoreMesh(core_axis_name="c", subcore_axis_name="s")

    @functools.partial(
        pl.kernel,
        out_type=jax.ShapeDtypeStruct((B0 * _SPAD, D), jnp.float32),
        mesh=mesh,
        scratch_types=[
            pltpu.VMEM((b_per_w * _IPAD,), jnp.int32),
            pltpu.VMEM((NBUF, _SPAD, D), jnp.float32),
            [pltpu.SemaphoreType.DMA] * NBUF,
            [pltpu.SemaphoreType.DMA] * NBUF,
        ],
    )
    def gather_kernel(table_hbm, idx_hbm, out_hbm, idx_v, rows_v,
                      gsems, ssems):
        wid = lax.axis_index("s") * NC + lax.axis_index("c")
        base = wid * b_per_w
        pltpu.sync_copy(idx_hbm.at[wid], idx_v)

        def gather(j, b):
            return pltpu.make_async_copy(
                table_hbm.at[idx_v.at[pl.ds(j * _IPAD, _SPAD)]], rows_v.at[b],
                gsems[b])

        def store(j, b):
            return pltpu.make_async_copy(
                rows_v.at[b], out_hbm.at[pl.ds((base + j) * _SPAD, _SPAD)],
                ssems[b])

        for b in range(NBUF):
            gather(b, b).start()

        def step(i, carry):
            j0 = i * NBUF
            for b in range(NBUF):
                j = j0 + b
                gather(j, b).wait()
                store(j, b).start()

                @pl.when(i < n_steps - 1)
                def _(j=j, b=b):
                    store(j, b).wait()
                    gather(j + NBUF, b).start()

            return carry

        lax.fori_loop(0, n_steps, step, 0)
        for b in range(NBUF):
            store(b_per_w - NBUF + b, b).wait()

    return gather_kernel


def kernel(token_ids, weight):
    B0, S = token_ids.shape
    D = weight.shape[1]
    NC, NS = _sc_geometry()
    NW = NC * NS
    idxp = jnp.pad(token_ids.astype(jnp.int32), ((0, 0), (0, _IPAD - S)))
    idxp = idxp.reshape(NW, (B0 // NW) * _IPAD)
    out = _make_gather(B0, S, D, NC, NS)(weight, idxp)
    return out.reshape(B0, _SPAD, D)[:, :S, :]


# speedup vs baseline: 7.8967x; 7.8967x over previous
import functools

import jax
import jax.numpy as jnp
from jax import lax
from jax.experimental import pallas as pl
from jax.experimental.pallas import tpu as pltpu
from jax.experimental.pallas import tpu_sc as plsc

_NBUF = 8
_GA = 6


def _sc_geometry():
    try:
        info = plsc.get_sparse_core_info()
        return info.num_cores, info.num_subcores
    except Exception:
        return 2, 16


@functools.lru_cache(maxsize=None)
def _make_gather(B0, S, D, NC, NS):
    NW = NC * NS
    n = B0 // NW
    assert n % _NBUF == 0 and n >= 2 * _NBUF
    mesh = plsc.VectorSubcoreMesh(core_axis_name="c", subcore_axis_name="s")

    @functools.partial(
        pl.kernel,
        out_type=jax.ShapeDtypeStruct((B0, S, D), jnp.float32),
        mesh=mesh,
        scratch_types=[
            pltpu.VMEM((n, S), jnp.int32),
            pltpu.VMEM((_NBUF, S, D), jnp.float32),
            [pltpu.SemaphoreType.DMA] * _NBUF,
            [pltpu.SemaphoreType.DMA] * _NBUF,
        ],
    )
    def gather_kernel(table_hbm, idx_hbm, out_hbm, idx_v, rows_v,
                      gsems, ssems):
        wid = lax.axis_index("s") * NC + lax.axis_index("c")
        base = wid * n
        pltpu.sync_copy(idx_hbm.at[pl.ds(base, n)], idx_v)

        def gather(j, slot):
            return pltpu.make_async_copy(
                table_hbm.at[idx_v.at[j]], rows_v.at[slot], gsems[slot])

        def store(j, slot):
            return pltpu.make_async_copy(
                rows_v.at[slot], out_hbm.at[base + j], ssems[slot])

        for j in range(_GA):
            gather(j, j).start()

        def step(i, carry):
            j0 = i * _NBUF
            for b in range(_NBUF):
                j = j0 + b
                gather(j, b).wait()
                store(j, b).start()
                nslot = (b + _GA) % _NBUF

                @pl.when((j + _GA < n) & (j + _GA >= _NBUF))
                def _(j=j, nslot=nslot):
                    store(j + _GA - _NBUF, nslot).wait()

                @pl.when(j + _GA < n)
                def _(j=j, nslot=nslot):
                    gather(j + _GA, nslot).start()

            return carry

        lax.fori_loop(0, n // _NBUF, step, 0)
        for j in range(n - _NBUF, n):
            store(j, j % _NBUF).wait()

    return gather_kernel


def kernel(token_ids, weight):
    B0, S = token_ids.shape
    D = weight.shape[1]
    NC, NS = _sc_geometry()
    return _make_gather(B0, S, D, NC, NS)(weight, token_ids.astype(jnp.int32))
